# baseline (speedup 1.0000x reference)
import functools
import math

import jax
import jax.numpy as jnp
from jax import lax
from jax.experimental import pallas as pl
from jax.experimental.pallas import tpu as pltpu
from jax.experimental.pallas import tpu_sc as plsc

N = 1024
E = 16384
DN = 256
H = 4
MH = 8
DK = 32

_EB = 1024
_NEB = E // _EB
_PB = 32768
_NPB = (N * N) // _PB
_IB = 128
_NIB = N // _IB

_INTERPRET = False


def _f32(x):
    return jnp.asarray(x, jnp.float32)


def _dotb(a, b):
    return jnp.dot(a.astype(jnp.bfloat16), b, preferred_element_type=jnp.float32)


def _a1_body(objc_ref, obja_ref, out_ref):
    cb = objc_ref[...]
    ca = obja_ref[...]
    dx = ca[0:1, :] - cb[:, 0:1]
    dy = ca[1:2, :] - cb[:, 1:2]
    dz = ca[2:3, :] - cb[:, 2:3]
    d = jnp.sqrt(dx * dx + dy * dy + dz * dz)
    out_ref[0, :, :] = dx
    out_ref[1, :, :] = dy
    out_ref[2, :, :] = dz
    out_ref[3, :, :] = d


def _norm_cols(h, ones_ref):
    m = jnp.dot(ones_ref[...], h, preferred_element_type=jnp.float32)
    s = jnp.dot(ones_ref[...], h * h, preferred_element_type=jnp.float32)
    inv = jax.lax.rsqrt(s - m * m + 1e-5)
    return (h - m) * inv


def _a2_body(w_ref, ones_ref, fc1w_ref, fc1b_ref, fc2w_ref, fc2b_ref,
             fc3w_ref, fc3b_ref, out_ref):
    w = w_ref[...]
    h = jnp.dot(fc1w_ref[...], w, preferred_element_type=jnp.float32)
    h = jax.nn.relu(h + fc1b_ref[...])
    h = _norm_cols(h, ones_ref)
    h = jnp.dot(fc2w_ref[...], h.astype(jnp.bfloat16),
                preferred_element_type=jnp.float32)
    h = jax.nn.relu(h + fc2b_ref[...])
    h = _norm_cols(h, ones_ref)
    o = jnp.dot(fc3w_ref[...], h.astype(jnp.bfloat16),
                preferred_element_type=jnp.float32)
    out_ref[...] = (o + fc3b_ref[...]).astype(jnp.bfloat16)


def _dist_weights(obj_center, params):
    objc = _f32(obj_center)
    obja = objc.T
    w4 = pl.pallas_call(
        _a1_body,
        grid=(_NIB,),
        in_specs=[
            pl.BlockSpec((_IB, 3), lambda i: (i, 0)),
            pl.BlockSpec((3, N), lambda i: (0, 0)),
        ],
        out_specs=pl.BlockSpec((4, _IB, N), lambda i: (0, i, 0)),
        out_shape=jax.ShapeDtypeStruct((4, N, N), jnp.float32),
        interpret=_INTERPRET,
    )(objc, obja)
    w4f = w4.reshape(4, N * N)

    p = params
    col = lambda v: v.reshape(-1, 1)
    full = lambda s: pl.BlockSpec(s, lambda i: tuple(0 for _ in s))
    fc2w = (p["fc2"]["w"] * p["ln1_g"][None, :]).astype(jnp.bfloat16)
    fc2b = col(p["fc2"]["b"]) + p["fc2"]["w"] @ col(p["ln1_b"])
    fc3w = (p["fc3"]["w"] * p["ln2_g"][None, :]).astype(jnp.bfloat16)
    fc3b = col(p["fc3"]["b"]) + p["fc3"]["w"] @ col(p["ln2_b"])
    ones = jnp.full((1, 32), 1.0 / 32.0, jnp.float32)
    dwf = pl.pallas_call(
        _a2_body,
        grid=(_NPB,),
        in_specs=[
            pl.BlockSpec((4, _PB), lambda i: (0, i)),
            full((1, 32)),
            full((32, 4)), full((32, 1)),
            full((32, 32)), full((32, 1)),
            full((8, 32)), full((8, 1)),
        ],
        out_specs=pl.BlockSpec((8, _PB), lambda i: (0, i)),
        out_shape=jax.ShapeDtypeStruct((8, N * N), jnp.bfloat16),
        interpret=_INTERPRET,
    )(w4f, ones, p["fc1"]["w"], col(p["fc1"]["b"]),
      fc2w, fc2b, fc3w, fc3b)
    return dwf.reshape(8, N, N)


def _attn_body(x_ref, wq_ref, bq_ref, wk_ref, bk_ref, wv_ref, bv_ref,
               wo_ref, bo_ref, dw_ref, out_ref):
    h = pl.program_id(0)
    x = x_ref[...]
    q = _dotb(x, wq_ref[0]) + bq_ref[0]
    k = _dotb(x, wk_ref[0]) + bk_ref[0]
    v = _dotb(x, wv_ref[0]) + bv_ref[0]
    att = lax.dot_general(q.astype(jnp.bfloat16), k.astype(jnp.bfloat16),
                          (((1,), (1,)), ((), ())),
                          preferred_element_type=jnp.float32)
    att = att * (1.0 / math.sqrt(DK)) + dw_ref[0].astype(jnp.float32)
    m = jnp.max(att, axis=-1, keepdims=True)
    e = jnp.exp(att - m)
    att = e / jnp.sum(e, axis=-1, keepdims=True)
    o = lax.dot_general(att.astype(jnp.bfloat16), v.astype(jnp.bfloat16),
                        (((1,), (0,)), ((), ())),
                        preferred_element_type=jnp.float32)
    contrib = _dotb(o, wo_ref[...])

    @pl.when(h == 0)
    def _():
        out_ref[...] = contrib + bo_ref[...]

    @pl.when(h > 0)
    def _():
        out_ref[...] = out_ref[...] + contrib


def _self_attn(x, lp, dw):
    full = lambda s: pl.BlockSpec(s, lambda h: tuple(0 for _ in s))
    hw = pl.BlockSpec((1, DN, DK), lambda h: (h, 0, 0))
    hb = pl.BlockSpec((1, 1, DK), lambda h: (h, 0, 0))
    perh = lambda p: p["w"].T.reshape(DN, MH, DK).transpose(1, 0, 2).astype(
        jnp.bfloat16)
    perb = lambda p: p["b"].reshape(MH, 1, DK)
    return pl.pallas_call(
        _attn_body,
        grid=(MH,),
        in_specs=[
            full((N, DN)),
            hw, hb, hw, hb, hw, hb,
            pl.BlockSpec((DK, DN), lambda h: (h, 0)),
            full((1, DN)),
            pl.BlockSpec((1, N, N), lambda h: (h, 0, 0)),
        ],
        out_specs=full((N, DN)),
        out_shape=jax.ShapeDtypeStruct((N, DN), jnp.float32),
        interpret=_INTERPRET,
    )(x, perh(lp["wq"]), perb(lp["wq"]),
      perh(lp["wk"]), perb(lp["wk"]),
      perh(lp["wv"]), perb(lp["wv"]),
      lp["wo"]["w"].T.astype(jnp.bfloat16), lp["wo"]["b"].reshape(1, DN), dw)


def _edge_body(xi_ref, xj_ref, ef_ref, ne1_ref, ne1b_ref, ne2_ref, ne2b_ref,
               pq_ref, pqb_ref, pe_ref, peb_ref, pv_ref, pvb_ref,
               a1_ref, a1b_ref, a2_ref, a2b_ref, unperm_ref,
               gcn_ref, prob_ref, msg_ref, *, relu_ef):
    xi16 = xi_ref[...].astype(jnp.bfloat16)
    xj16 = xj_ref[...].astype(jnp.bfloat16)
    e = ef_ref[...]
    if relu_ef:
        e = jax.nn.relu(e)
    ne1 = ne1_ref[...]
    e16 = e.astype(jnp.bfloat16)
    g1 = (jnp.dot(xi16, ne1[0:DN], preferred_element_type=jnp.float32)
          + jnp.dot(e16, ne1[DN:2 * DN], preferred_element_type=jnp.float32)
          + jnp.dot(xj16, ne1[2 * DN:], preferred_element_type=jnp.float32)
          + ne1b_ref[...])
    g1 = jax.nn.relu(g1)
    gcn_ref[...] = (_dotb(g1, ne2_ref[...])
                    + ne2b_ref[...]).astype(gcn_ref.dtype)
    q = jax.nn.relu(jnp.dot(xi16, pq_ref[...],
                            preferred_element_type=jnp.float32)
                    + pqb_ref[...])
    ed = jax.nn.relu(jnp.dot(e16, pe_ref[...],
                             preferred_element_type=jnp.float32)
                     + peb_ref[...])
    val = jax.nn.relu(jnp.dot(xj16, pv_ref[...],
                              preferred_element_type=jnp.float32)
                      + pvb_ref[...])
    msg_parts = []
    for h in range(H):
        sl = slice(h * 64, (h + 1) * 64)
        in_h = jnp.concatenate([q[:, sl], ed[:, sl]], axis=1)
        h1 = jax.nn.relu(_dotb(in_h, a1_ref[...]) + a1b_ref[...])
        logits = _dotb(h1, a2_ref[...]) + a2b_ref[...]
        m = jnp.max(logits, axis=-1, keepdims=True)
        ex = jnp.exp(logits - m)
        p = ex / jnp.sum(ex, axis=-1, keepdims=True)
        msg_parts.append(p)
    pcat = jnp.concatenate(msg_parts, axis=1)
    msg_ref[...] = pcat * val
    prob_ref[...] = jnp.dot(pcat.astype(jnp.bfloat16), unperm_ref[...],
                            preferred_element_type=jnp.float32)


def _edge_stage(gathered, ef, wp, relu_ef, gcn_dtype):
    full = lambda s: pl.BlockSpec(s, lambda i: tuple(0 for _ in s))
    eb = lambda c: pl.BlockSpec((_EB, c), lambda i: (i, 0))
    return pl.pallas_call(
        functools.partial(_edge_body, relu_ef=relu_ef),
        grid=(_NEB,),
        in_specs=[
            pl.BlockSpec((_EB, DN), lambda i: (i, 0)),
            pl.BlockSpec((_EB, DN), lambda i: (i + _NEB, 0)),
            eb(DN),
            full((3 * DN, 2 * DN)), full((1, 2 * DN)),
            full((2 * DN, DN)), full((1, DN)),
            full((DN, DN)), full((1, DN)),
            full((DN, DN)), full((1, DN)),
            full((DN, DN)), full((1, DN)),
            full((128, 128)), full((1, 128)),
            full((128, 64)), full((1, 64)),
            full((DN, DN)),
        ],
        out_specs=(
            eb(DN),
            eb(DN),
            eb(DN),
        ),
        out_shape=(
            jax.ShapeDtypeStruct((E, DN), gcn_dtype),
            jax.ShapeDtypeStruct((E, DN), jnp.float32),
            jax.ShapeDtypeStruct((E, DN), jnp.float32),
        ),
        interpret=_INTERPRET,
    )(gathered, gathered, ef, *wp)


_NACC = 32


def _seg_body(idx_ref, msg_ref, x_ref, p1a_ref, p1b_ref, p1bias_ref,
              p2_ref, p2b_ref, out_ref, *agg_refs, apply_relu):
    i = pl.program_id(0)

    @pl.when(i == 0)
    def _():
        for r in agg_refs:
            r[...] = jnp.zeros_like(r)

    base = i * _EB

    def body(g, carry):
        e = g * _NACC
        for k in range(_NACC):
            n = idx_ref[base + e + k]
            r = agg_refs[k]
            cur = r[pl.ds(n, 1), :]
            row = msg_ref[pl.ds(e + k, 1), :]
            r[pl.ds(n, 1), :] = jnp.maximum(cur, row)
        return carry

    lax.fori_loop(0, _EB // _NACC, body, 0)

    @pl.when(i == _NEB - 1)
    def _():
        x = x_ref[...]
        a = agg_refs[0][...]
        for k in range(1, _NACC):
            a = jnp.maximum(a, agg_refs[k][...])
        h = (jnp.dot(x, p1a_ref[...], preferred_element_type=jnp.float32)
             + jnp.dot(a, p1b_ref[...], preferred_element_type=jnp.float32)
             + p1bias_ref[...])
        h = jax.nn.relu(h)
        o = jnp.dot(h, p2_ref[...], preferred_element_type=jnp.float32) + p2b_ref[...]
        if apply_relu:
            o = jax.nn.relu(o)
        out_ref[...] = o


def _segment_stage(idx_i, msg, x, p1a, p1b, p1bias, p2, p2b, apply_relu):
    full = lambda s: pl.BlockSpec(s, lambda i, *_: tuple(0 for _ in s))
    grid_spec = pltpu.PrefetchScalarGridSpec(
        num_scalar_prefetch=1,
        grid=(_NEB,),
        in_specs=[
            pl.BlockSpec((_EB, DN), lambda i, *_: (i, 0)),
            full((N, DN)),
            full((DN, 2 * DN)), full((DN, 2 * DN)), full((1, 2 * DN)),
            full((2 * DN, DN)), full((1, DN)),
        ],
        out_specs=full((N, DN)),
        scratch_shapes=[pltpu.VMEM((N, DN), jnp.float32)
                        for _ in range(_NACC)],
    )
    return pl.pallas_call(
        functools.partial(_seg_body, apply_relu=apply_relu),
        grid_spec=grid_spec,
        out_shape=jax.ShapeDtypeStruct((N, DN), jnp.float32),
        interpret=_INTERPRET,
    )(idx_i, msg, x, p1a, p1b, p1bias, p2, p2b)


_GW = 128
_NWORK = 32


def _sc_gather(table, idx_all):
    b_total = idx_all.shape[0]
    bpw = b_total // _NWORK
    nch = bpw // _GW
    mesh = plsc.VectorSubcoreMesh(core_axis_name="c", subcore_axis_name="s")

    @functools.partial(
        pl.kernel, mesh=mesh,
        out_type=jax.ShapeDtypeStruct((b_total, DN), jnp.float32),
        scratch_types=[
            pltpu.VMEM((_GW,), jnp.int32),
            pltpu.VMEM((_GW, DN), jnp.float32),
            pltpu.VMEM((_GW,), jnp.int32),
            pltpu.VMEM((_GW, DN), jnp.float32),
            pltpu.SemaphoreType.DMA,
            pltpu.SemaphoreType.DMA,
        ],
    )
    def k(table_hbm, idx_hbm, out_hbm, idx0, rows0, idx1, rows1, sem0, sem1):
        wid = lax.axis_index("s") * 2 + lax.axis_index("c")
        base = wid * bpw

        @pl.loop(0, nch // 2)
        def _(c):
            b0 = base + (2 * c) * _GW
            b1 = b0 + _GW
            pltpu.sync_copy(idx_hbm.at[pl.ds(b0, _GW)], idx0)
            cp0 = pltpu.async_copy(table_hbm.at[idx0], rows0, sem0)
            pltpu.sync_copy(idx_hbm.at[pl.ds(b1, _GW)], idx1)
            cp1 = pltpu.async_copy(table_hbm.at[idx1], rows1, sem1)
            cp0.wait()
            pltpu.sync_copy(rows0, out_hbm.at[pl.ds(b0, _GW)])
            cp1.wait()
            pltpu.sync_copy(rows1, out_hbm.at[pl.ds(b1, _GW)])

    return k(table, idx_all)


def _perm_cols():
    return jnp.asarray([c * 4 + h for h in range(H) for c in range(64)],
                       jnp.int32)


def _edge_weights(lp, perm):
    t = lambda p: p["w"].T.astype(jnp.bfloat16)
    row = lambda p: p["b"].reshape(1, -1)
    pqT = t(lp["pq"])[:, perm]
    pqb = row(lp["pq"])[:, perm]
    peT = t(lp["pe"])[:, perm]
    peb = row(lp["pe"])[:, perm]
    pvT = t(lp["pv"])[:, perm]
    pvb = row(lp["pv"])[:, perm]
    unperm = jnp.eye(DN, dtype=jnp.bfloat16)[perm]
    return (t(lp["ne1"]), row(lp["ne1"]), t(lp["ne2"]), row(lp["ne2"]),
            pqT, pqb, peT, peb, pvT, pvb,
            t(lp["att1"]), row(lp["att1"]), t(lp["att2"]), row(lp["att2"]),
            unperm)


def kernel(node_feature, edge_feature, edges_indices, obj_center, batch_ids,
           params):
    del batch_ids
    perm = _perm_cols()
    dw = _dist_weights(obj_center, params)
    idx_i = edges_indices[0]
    idx_all = edges_indices.reshape(2 * E)

    nf = _f32(node_feature)
    ef = _f32(edge_feature)
    probs = []
    nl = len(params["layers"])
    for i, lp in enumerate(params["layers"]):
        inner = i < nl - 1
        nf = _self_attn(nf, lp, dw)
        gathered = _sc_gather(nf, idx_all)
        wp = _edge_weights(lp, perm)
        gcn_e, prob_flat, msg = _edge_stage(gathered, ef, wp,
                                            relu_ef=(i > 0),
                                            gcn_dtype=jnp.float32)
        probs.append(prob_flat.reshape(E, 64, H))
        p1 = lp["pr1"]["w"].T
        p1a = p1[:DN]
        p1b = p1[DN:][perm]
        nf = _segment_stage(idx_i, msg, nf, p1a, p1b,
                            lp["pr1"]["b"].reshape(1, -1),
                            lp["pr2"]["w"].T, lp["pr2"]["b"].reshape(1, -1),
                            apply_relu=inner)
        ef = gcn_e
    return nf, ef, jnp.stack(probs)


# speedup vs baseline: 1.0012x; 1.0012x over previous
import functools
import math

import jax
import jax.numpy as jnp
from jax import lax
from jax.experimental import pallas as pl
from jax.experimental.pallas import tpu as pltpu
from jax.experimental.pallas import tpu_sc as plsc

N = 1024
E = 16384
DN = 256
H = 4
MH = 8
DK = 32

_EB = 1024
_NEB = E // _EB
_PB = 32768
_NPB = (N * N) // _PB
_IB = 128
_NIB = N // _IB


def _f32(x):
    return jnp.asarray(x, jnp.float32)


def _dotb(a, b):
    return jnp.dot(a.astype(jnp.bfloat16), b, preferred_element_type=jnp.float32)


def _a1_body(objc_ref, obja_ref, out_ref):
    cb = objc_ref[...]
    ca = obja_ref[...]
    dx = ca[0:1, :] - cb[:, 0:1]
    dy = ca[1:2, :] - cb[:, 1:2]
    dz = ca[2:3, :] - cb[:, 2:3]
    d = jnp.sqrt(dx * dx + dy * dy + dz * dz)
    out_ref[0, :, :] = dx
    out_ref[1, :, :] = dy
    out_ref[2, :, :] = dz
    out_ref[3, :, :] = d


def _norm_cols(h, ones_ref):
    m = jnp.dot(ones_ref[...], h, preferred_element_type=jnp.float32)
    s = jnp.dot(ones_ref[...], h * h, preferred_element_type=jnp.float32)
    inv = jax.lax.rsqrt(s - m * m + 1e-5)
    return (h - m) * inv


def _a2_body(w_ref, ones_ref, fc1w_ref, fc1b_ref, fc2w_ref, fc2b_ref,
             fc3w_ref, fc3b_ref, out_ref):
    w = w_ref[...]
    h = jnp.dot(fc1w_ref[...], w, preferred_element_type=jnp.float32)
    h = jax.nn.relu(h + fc1b_ref[...])
    h = _norm_cols(h, ones_ref)
    h = jnp.dot(fc2w_ref[...], h.astype(jnp.bfloat16),
                preferred_element_type=jnp.float32)
    h = jax.nn.relu(h + fc2b_ref[...])
    h = _norm_cols(h, ones_ref)
    o = jnp.dot(fc3w_ref[...], h.astype(jnp.bfloat16),
                preferred_element_type=jnp.float32)
    out_ref[...] = (o + fc3b_ref[...]).astype(jnp.bfloat16)


def _dist_weights(obj_center, params):
    objc = _f32(obj_center)
    obja = objc.T
    w4 = pl.pallas_call(
        _a1_body,
        grid=(_NIB,),
        in_specs=[
            pl.BlockSpec((_IB, 3), lambda i: (i, 0)),
            pl.BlockSpec((3, N), lambda i: (0, 0)),
        ],
        out_specs=pl.BlockSpec((4, _IB, N), lambda i: (0, i, 0)),
        out_shape=jax.ShapeDtypeStruct((4, N, N), jnp.float32),
    )(objc, obja)
    w4f = w4.reshape(4, N * N)

    p = params
    col = lambda v: v.reshape(-1, 1)
    full = lambda s: pl.BlockSpec(s, lambda i: tuple(0 for _ in s))
    fc2w = (p["fc2"]["w"] * p["ln1_g"][None, :]).astype(jnp.bfloat16)
    fc2b = col(p["fc2"]["b"]) + p["fc2"]["w"] @ col(p["ln1_b"])
    fc3w = (p["fc3"]["w"] * p["ln2_g"][None, :]).astype(jnp.bfloat16)
    fc3b = col(p["fc3"]["b"]) + p["fc3"]["w"] @ col(p["ln2_b"])
    ones = jnp.full((1, 32), 1.0 / 32.0, jnp.float32)
    dwf = pl.pallas_call(
        _a2_body,
        grid=(_NPB,),
        in_specs=[
            pl.BlockSpec((4, _PB), lambda i: (0, i)),
            full((1, 32)),
            full((32, 4)), full((32, 1)),
            full((32, 32)), full((32, 1)),
            full((8, 32)), full((8, 1)),
        ],
        out_specs=pl.BlockSpec((8, _PB), lambda i: (0, i)),
        out_shape=jax.ShapeDtypeStruct((8, N * N), jnp.bfloat16),
    )(w4f, ones, p["fc1"]["w"], col(p["fc1"]["b"]),
      fc2w, fc2b, fc3w, fc3b)
    return dwf.reshape(8, N, N)


def _attn_body(x_ref, wq_ref, bq_ref, wk_ref, bk_ref, wv_ref, bv_ref,
               wo_ref, bo_ref, dw_ref, out_ref):
    h = pl.program_id(0)
    x = x_ref[...]
    q = _dotb(x, wq_ref[0]) + bq_ref[0]
    k = _dotb(x, wk_ref[0]) + bk_ref[0]
    v = _dotb(x, wv_ref[0]) + bv_ref[0]
    att = lax.dot_general(q.astype(jnp.bfloat16), k.astype(jnp.bfloat16),
                          (((1,), (1,)), ((), ())),
                          preferred_element_type=jnp.float32)
    att = att * (1.0 / math.sqrt(DK)) + dw_ref[0].astype(jnp.float32)
    m = jnp.max(att, axis=-1, keepdims=True)
    e = jnp.exp(att - m)
    att = e / jnp.sum(e, axis=-1, keepdims=True)
    o = lax.dot_general(att.astype(jnp.bfloat16), v.astype(jnp.bfloat16),
                        (((1,), (0,)), ((), ())),
                        preferred_element_type=jnp.float32)
    contrib = _dotb(o, wo_ref[...])

    @pl.when(h == 0)
    def _():
        out_ref[...] = contrib + bo_ref[...]

    @pl.when(h > 0)
    def _():
        out_ref[...] = out_ref[...] + contrib


def _self_attn(x, lp, dw):
    full = lambda s: pl.BlockSpec(s, lambda h: tuple(0 for _ in s))
    hw = pl.BlockSpec((1, DN, DK), lambda h: (h, 0, 0))
    hb = pl.BlockSpec((1, 1, DK), lambda h: (h, 0, 0))
    perh = lambda p: p["w"].T.reshape(DN, MH, DK).transpose(1, 0, 2).astype(
        jnp.bfloat16)
    perb = lambda p: p["b"].reshape(MH, 1, DK)
    return pl.pallas_call(
        _attn_body,
        grid=(MH,),
        in_specs=[
            full((N, DN)),
            hw, hb, hw, hb, hw, hb,
            pl.BlockSpec((DK, DN), lambda h: (h, 0)),
            full((1, DN)),
            pl.BlockSpec((1, N, N), lambda h: (h, 0, 0)),
        ],
        out_specs=full((N, DN)),
        out_shape=jax.ShapeDtypeStruct((N, DN), jnp.float32),
    )(x, perh(lp["wq"]), perb(lp["wq"]),
      perh(lp["wk"]), perb(lp["wk"]),
      perh(lp["wv"]), perb(lp["wv"]),
      lp["wo"]["w"].T.astype(jnp.bfloat16), lp["wo"]["b"].reshape(1, DN), dw)


def _edge_body(xi_ref, xj_ref, ef_ref, ne1_ref, ne1b_ref, ne2_ref, ne2b_ref,
               pq_ref, pqb_ref, pe_ref, peb_ref, pv_ref, pvb_ref,
               a1_ref, a1b_ref, a2_ref, a2b_ref, unperm_ref,
               gcn_ref, prob_ref, msg_ref, *, relu_ef):
    xi16 = xi_ref[...].astype(jnp.bfloat16)
    xj16 = xj_ref[...].astype(jnp.bfloat16)
    e = ef_ref[...]
    if relu_ef:
        e = jax.nn.relu(e)
    ne1 = ne1_ref[...]
    e16 = e.astype(jnp.bfloat16)
    g1 = (jnp.dot(xi16, ne1[0:DN], preferred_element_type=jnp.float32)
          + jnp.dot(e16, ne1[DN:2 * DN], preferred_element_type=jnp.float32)
          + jnp.dot(xj16, ne1[2 * DN:], preferred_element_type=jnp.float32)
          + ne1b_ref[...])
    g1 = jax.nn.relu(g1)
    gcn_ref[...] = (_dotb(g1, ne2_ref[...])
                    + ne2b_ref[...]).astype(gcn_ref.dtype)
    q = jax.nn.relu(jnp.dot(xi16, pq_ref[...],
                            preferred_element_type=jnp.float32)
                    + pqb_ref[...])
    ed = jax.nn.relu(jnp.dot(e16, pe_ref[...],
                             preferred_element_type=jnp.float32)
                     + peb_ref[...])
    val = jax.nn.relu(jnp.dot(xj16, pv_ref[...],
                              preferred_element_type=jnp.float32)
                      + pvb_ref[...])
    msg_parts = []
    for h in range(H):
        sl = slice(h * 64, (h + 1) * 64)
        in_h = jnp.concatenate([q[:, sl], ed[:, sl]], axis=1)
        h1 = jax.nn.relu(_dotb(in_h, a1_ref[...]) + a1b_ref[...])
        logits = _dotb(h1, a2_ref[...]) + a2b_ref[...]
        m = jnp.max(logits, axis=-1, keepdims=True)
        ex = jnp.exp(logits - m)
        p = ex / jnp.sum(ex, axis=-1, keepdims=True)
        msg_parts.append(p)
    pcat = jnp.concatenate(msg_parts, axis=1)
    msg_ref[...] = pcat * val
    prob_ref[...] = jnp.dot(pcat.astype(jnp.bfloat16), unperm_ref[...],
                            preferred_element_type=jnp.float32)


def _edge_stage(gathered, ef, wp, relu_ef, gcn_dtype):
    full = lambda s: pl.BlockSpec(s, lambda i: tuple(0 for _ in s))
    eb = lambda c: pl.BlockSpec((_EB, c), lambda i: (i, 0))
    return pl.pallas_call(
        functools.partial(_edge_body, relu_ef=relu_ef),
        grid=(_NEB,),
        in_specs=[
            pl.BlockSpec((_EB, DN), lambda i: (i, 0)),
            pl.BlockSpec((_EB, DN), lambda i: (i + _NEB, 0)),
            eb(DN),
            full((3 * DN, 2 * DN)), full((1, 2 * DN)),
            full((2 * DN, DN)), full((1, DN)),
            full((DN, DN)), full((1, DN)),
            full((DN, DN)), full((1, DN)),
            full((DN, DN)), full((1, DN)),
            full((128, 128)), full((1, 128)),
            full((128, 64)), full((1, 64)),
            full((DN, DN)),
        ],
        out_specs=(
            eb(DN),
            eb(DN),
            eb(DN),
        ),
        out_shape=(
            jax.ShapeDtypeStruct((E, DN), gcn_dtype),
            jax.ShapeDtypeStruct((E, DN), jnp.float32),
            jax.ShapeDtypeStruct((E, DN), jnp.float32),
        ),
    )(gathered, gathered, ef, *wp)


_NACC = 16


def _seg_body(idx_ref, msg_ref, x_ref, p1a_ref, p1b_ref, p1bias_ref,
              p2_ref, p2b_ref, out_ref, *agg_refs, apply_relu):
    i = pl.program_id(0)

    @pl.when(i == 0)
    def _():
        for r in agg_refs:
            r[...] = jnp.zeros_like(r)

    base = i * _EB

    def body(g, carry):
        e = g * _NACC
        for k in range(_NACC):
            n = idx_ref[base + e + k]
            r = agg_refs[k]
            cur = r[pl.ds(n, 1), :]
            row = msg_ref[pl.ds(e + k, 1), :]
            r[pl.ds(n, 1), :] = jnp.maximum(cur, row)
        return carry

    lax.fori_loop(0, _EB // _NACC, body, 0)

    @pl.when(i == _NEB - 1)
    def _():
        x = x_ref[...]
        a = agg_refs[0][...]
        for k in range(1, _NACC):
            a = jnp.maximum(a, agg_refs[k][...])
        h = (jnp.dot(x, p1a_ref[...], preferred_element_type=jnp.float32)
             + jnp.dot(a, p1b_ref[...], preferred_element_type=jnp.float32)
             + p1bias_ref[...])
        h = jax.nn.relu(h)
        o = jnp.dot(h, p2_ref[...], preferred_element_type=jnp.float32) + p2b_ref[...]
        if apply_relu:
            o = jax.nn.relu(o)
        out_ref[...] = o


def _segment_stage(idx_i, msg, x, p1a, p1b, p1bias, p2, p2b, apply_relu):
    full = lambda s: pl.BlockSpec(s, lambda i, *_: tuple(0 for _ in s))
    grid_spec = pltpu.PrefetchScalarGridSpec(
        num_scalar_prefetch=1,
        grid=(_NEB,),
        in_specs=[
            pl.BlockSpec((_EB, DN), lambda i, *_: (i, 0)),
            full((N, DN)),
            full((DN, 2 * DN)), full((DN, 2 * DN)), full((1, 2 * DN)),
            full((2 * DN, DN)), full((1, DN)),
        ],
        out_specs=full((N, DN)),
        scratch_shapes=[pltpu.VMEM((N, DN), jnp.float32)
                        for _ in range(_NACC)],
    )
    return pl.pallas_call(
        functools.partial(_seg_body, apply_relu=apply_relu),
        grid_spec=grid_spec,
        out_shape=jax.ShapeDtypeStruct((N, DN), jnp.float32),
    )(idx_i, msg, x, p1a, p1b, p1bias, p2, p2b)


_GW = 128
_NWORK = 32


def _sc_gather(table, idx_all):
    b_total = idx_all.shape[0]
    bpw = b_total // _NWORK
    nch = bpw // _GW
    mesh = plsc.VectorSubcoreMesh(core_axis_name="c", subcore_axis_name="s")

    @functools.partial(
        pl.kernel, mesh=mesh,
        out_type=jax.ShapeDtypeStruct((b_total, DN), jnp.float32),
        scratch_types=[
            pltpu.VMEM((_GW,), jnp.int32),
            pltpu.VMEM((_GW, DN), jnp.float32),
            pltpu.VMEM((_GW,), jnp.int32),
            pltpu.VMEM((_GW, DN), jnp.float32),
            pltpu.SemaphoreType.DMA,
            pltpu.SemaphoreType.DMA,
        ],
    )
    def k(table_hbm, idx_hbm, out_hbm, idx0, rows0, idx1, rows1, sem0, sem1):
        wid = lax.axis_index("s") * 2 + lax.axis_index("c")
        base = wid * bpw

        @pl.loop(0, nch // 2)
        def _(c):
            b0 = base + (2 * c) * _GW
            b1 = b0 + _GW
            pltpu.sync_copy(idx_hbm.at[pl.ds(b0, _GW)], idx0)
            cp0 = pltpu.async_copy(table_hbm.at[idx0], rows0, sem0)
            pltpu.sync_copy(idx_hbm.at[pl.ds(b1, _GW)], idx1)
            cp1 = pltpu.async_copy(table_hbm.at[idx1], rows1, sem1)
            cp0.wait()
            pltpu.sync_copy(rows0, out_hbm.at[pl.ds(b0, _GW)])
            cp1.wait()
            pltpu.sync_copy(rows1, out_hbm.at[pl.ds(b1, _GW)])

    return k(table, idx_all)


def _perm_cols():
    return jnp.asarray([c * 4 + h for h in range(H) for c in range(64)],
                       jnp.int32)


def _edge_weights(lp, perm):
    t = lambda p: p["w"].T.astype(jnp.bfloat16)
    row = lambda p: p["b"].reshape(1, -1)
    pqT = t(lp["pq"])[:, perm]
    pqb = row(lp["pq"])[:, perm]
    peT = t(lp["pe"])[:, perm]
    peb = row(lp["pe"])[:, perm]
    pvT = t(lp["pv"])[:, perm]
    pvb = row(lp["pv"])[:, perm]
    unperm = jnp.eye(DN, dtype=jnp.bfloat16)[perm]
    return (t(lp["ne1"]), row(lp["ne1"]), t(lp["ne2"]), row(lp["ne2"]),
            pqT, pqb, peT, peb, pvT, pvb,
            t(lp["att1"]), row(lp["att1"]), t(lp["att2"]), row(lp["att2"]),
            unperm)


def kernel(node_feature, edge_feature, edges_indices, obj_center, batch_ids,
           params):
    del batch_ids
    perm = _perm_cols()
    dw = _dist_weights(obj_center, params)
    idx_i = edges_indices[0]
    idx_all = edges_indices.reshape(2 * E)

    nf = _f32(node_feature)
    ef = _f32(edge_feature)
    probs = []
    nl = len(params["layers"])
    for i, lp in enumerate(params["layers"]):
        inner = i < nl - 1
        nf = _self_attn(nf, lp, dw)
        gathered = _sc_gather(nf, idx_all)
        wp = _edge_weights(lp, perm)
        gcn_e, prob_flat, msg = _edge_stage(gathered, ef, wp,
                                            relu_ef=(i > 0),
                                            gcn_dtype=jnp.float32)
        probs.append(prob_flat.reshape(E, 64, H))
        p1 = lp["pr1"]["w"].T
        p1a = p1[:DN]
        p1b = p1[DN:][perm]
        nf = _segment_stage(idx_i, msg, nf, p1a, p1b,
                            lp["pr1"]["b"].reshape(1, -1),
                            lp["pr2"]["w"].T, lp["pr2"]["b"].reshape(1, -1),
                            apply_relu=inner)
        ef = gcn_e
    return nf, ef, jnp.stack(probs)
